# baseline (device time: 266328 ns/iter reference)
import jax
import jax.numpy as jnp
from jax import lax
from jax.experimental import pallas as pl
from jax.experimental.pallas import tpu as pltpu

_BM = 256
_BN = 256
_NCH = 16
_NB = 4
_NH = 16
_HD = 128
_NK = _NCH * _NB


def _body(o_in, wo_ref, o_ref, wo_bufs, a3_buf, a_buf, send_buf,
          wo_sems, c_sems, a_sems,
          y_s, y_r, x_s, x_r, z_s, z_r, f_s, f_r):
    my_x = lax.axis_index("x")
    my_y = lax.axis_index("y")
    my_z = lax.axis_index("z")
    nbr_y = (my_x, 1 - my_y, my_z)
    nbr_x = (1 - my_x, my_y, my_z)
    nbr_z = (my_x, my_y, 1 - my_z)
    is_holder = my_y == my_x
    rows_base = 512 * my_x + _BM * my_z

    def contig_start(b, slot):
        pltpu.make_async_copy(
            o_in.at[b, pl.ds(rows_base, _BM), :, :], a3_buf.at[slot],
            c_sems.at[slot],
        ).start()

    def contig_wait(slot):
        pltpu.make_async_copy(
            o_in.at[0, pl.ds(0, _BM), :, :], a3_buf.at[slot],
            c_sems.at[slot],
        ).wait()

    def transpose(slot, b):
        cps = [
            pltpu.make_async_copy(
                a3_buf.at[slot, :, h, :],
                a_buf.at[b, :, pl.ds(h * _HD, _HD)],
                a_sems.at[slot],
            )
            for h in range(_NH)
        ]
        for cp in cps:
            cp.start()
        for cp in cps:
            cp.wait()

    def slc(b, ch, q):
        return o_ref.at[b, q, :, pl.ds(ch * _BN, _BN)]

    def rd(src, dst, ssem, rsem, k, dev):
        return pltpu.make_async_remote_copy(
            src_ref=src, dst_ref=dst, send_sem=ssem.at[k], recv_sem=rsem.at[k],
            device_id=dev, device_id_type=pl.DeviceIdType.MESH,
        )

    def rd_y(k, slot, b, ch):
        return rd(send_buf.at[slot], slc(b, ch, my_z), y_s, y_r, k, nbr_y)

    def rd_x(k, b, ch):
        return rd(slc(b, ch, my_z), slc(b, ch, my_z), x_s, x_r, k, nbr_x)

    def rd_z(k, b, ch):
        return rd(slc(b, ch, my_z), slc(b, ch, my_z), z_s, z_r, k, nbr_z)

    def rd_z_recv(k, b, ch):
        return rd(slc(b, ch, my_z), slc(b, ch, 1 - my_z), z_s, z_r, k, nbr_z)

    def rd_f(k, b, ch):
        return rd(slc(b, ch, my_z), slc(b, ch, my_z), f_s, f_r, k, nbr_z)

    def rd_f_recv(k, b, ch):
        return rd(slc(b, ch, my_z), slc(b, ch, 1 - my_z), f_s, f_r, k, nbr_z)

    def wo_start(ch, par):
        pltpu.make_async_copy(
            wo_ref.at[:, pl.ds(ch * _BN, _BN)], wo_bufs.at[par],
            wo_sems.at[par],
        ).start()

    def wo_wait(ch, par):
        pltpu.make_async_copy(
            wo_ref.at[:, pl.ds(ch * _BN, _BN)], wo_bufs.at[par],
            wo_sems.at[par],
        ).wait()

    wo_start(0, 0)
    contig_start(0, 0)
    barrier = pltpu.get_barrier_semaphore()
    for nbr in (nbr_y, nbr_x, nbr_z):
        pl.semaphore_signal(barrier, inc=1, device_id=nbr,
                            device_id_type=pl.DeviceIdType.MESH)
    pl.semaphore_wait(barrier, 3)

    for ch in range(_NCH):
        par = ch % 2
        wo_wait(ch, par)
        if ch + 1 < _NCH:
            wo_start(ch + 1, 1 - par)

        for b in range(_NB):
            k = ch * _NB + b
            slot = k % 2
            if ch == 0:
                if b + 1 < _NB:
                    contig_start(b + 1, (b + 1) % 2)
                contig_wait(b % 2)
                transpose(b % 2, b)
            v = jnp.dot(a_buf[b], wo_bufs[par],
                        preferred_element_type=jnp.float32)

            @pl.when(jnp.logical_not(is_holder))
            def _(v=v, k=k, slot=slot, b=b, ch=ch):
                if k >= 2:
                    rd_y(k - 2, slot, b, ch).wait_send()
                send_buf[slot] = v
                rd_y(k, slot, b, ch).start()
                if k >= 2:
                    j = k - 2
                    jb, jch = j % _NB, j // _NB
                    rd_x(j, jb, jch).wait_recv()
                    rd_f(j, jb, jch).start()

            @pl.when(is_holder)
            def _(v=v, k=k, slot=slot, b=b, ch=ch):
                rd_y(k, slot, b, ch).wait_recv()
                o_ref[b, my_z, :, pl.ds(ch * _BN, _BN)] = (
                    o_ref[b, my_z, :, pl.ds(ch * _BN, _BN)] + v
                )
                rd_x(k, b, ch).start()
                rd_z(k, b, ch).start()

    for k in (_NK - 2, _NK - 1):
        b, ch = k % _NB, k // _NB

        @pl.when(jnp.logical_not(is_holder))
        def _(k=k, b=b, ch=ch):
            rd_x(k, b, ch).wait_recv()
            rd_f(k, b, ch).start()

    for k in range(_NK):
        b, ch = k % _NB, k // _NB

        @pl.when(is_holder)
        def _(k=k, b=b, ch=ch):
            rd_z_recv(k, b, ch).wait_recv()

        @pl.when(jnp.logical_not(is_holder))
        def _(k=k, b=b, ch=ch):
            rd_f_recv(k, b, ch).wait_recv()

    @pl.when(is_holder)
    def _():
        for k in range(_NK):
            b, ch = k % _NB, k // _NB
            rd_x(k, b, ch).wait_send()
            rd_z(k, b, ch).wait_send()

    @pl.when(jnp.logical_not(is_holder))
    def _():
        for k in (_NK - 2, _NK - 1):
            rd_y(k, k % 2, k % _NB, k // _NB).wait_send()
        for k in range(_NK):
            b, ch = k % _NB, k // _NB
            rd_f(k, b, ch).wait_send()

    def _exit(second_barrier):
        for nbr in (nbr_y, nbr_x, nbr_z):
            pl.semaphore_signal(second_barrier, inc=1, device_id=nbr,
                                device_id_type=pl.DeviceIdType.MESH)
        pl.semaphore_wait(second_barrier, 3)

    pl.run_scoped(_exit, second_barrier=pltpu.SemaphoreType.REGULAR)


def kernel(O, Wo):
    B, S, Hl, D = O.shape
    N = Wo.shape[1]
    out = pl.pallas_call(
        _body,
        out_shape=jax.ShapeDtypeStruct((B, 2, _BM, N), jnp.float32),
        in_specs=[
            pl.BlockSpec(memory_space=pl.ANY),
            pl.BlockSpec(memory_space=pl.ANY),
        ],
        out_specs=pl.BlockSpec(memory_space=pltpu.VMEM),
        scratch_shapes=[
            pltpu.VMEM((2, Hl * D, _BN), jnp.float32),
            pltpu.VMEM((2, _BM, Hl, D), jnp.float32),
            pltpu.VMEM((B, _BM, Hl * D), jnp.float32),
            pltpu.VMEM((2, _BM, _BN), jnp.float32),
            pltpu.SemaphoreType.DMA((2,)),
            pltpu.SemaphoreType.DMA((2,)),
            pltpu.SemaphoreType.DMA((2,)),
        ] + [pltpu.SemaphoreType.DMA((_NK,)) for _ in range(8)],
        compiler_params=pltpu.CompilerParams(
            collective_id=0,
            vmem_limit_bytes=62 * 1024 * 1024,
        ),
    )(O, Wo)
    return out.reshape(B, 2 * _BM, N)


# device time: 241493 ns/iter; 1.1028x vs baseline; 1.1028x over previous
import jax
import jax.numpy as jnp
from jax import lax
from jax.experimental import pallas as pl
from jax.experimental.pallas import tpu as pltpu

_BM = 256
_BN = 512
_NCH = 8
_NB = 4
_NH = 16
_HD = 128
_NK = _NCH * _NB


def _body(o_in, wo_ref, o_ref, wo_bufs, a3_buf, a_buf, send_buf,
          wo_sems, c_sems, a_sems,
          y_s, y_r, x_s, x_r, z_s, z_r, f_s, f_r):
    my_x = lax.axis_index("x")
    my_y = lax.axis_index("y")
    my_z = lax.axis_index("z")
    nbr_y = (my_x, 1 - my_y, my_z)
    nbr_x = (1 - my_x, my_y, my_z)
    nbr_z = (my_x, my_y, 1 - my_z)
    is_holder = my_y == my_x
    rows_base = 512 * my_x + _BM * my_z

    def contig_start(b, slot):
        pltpu.make_async_copy(
            o_in.at[b, pl.ds(rows_base, _BM), :, :], a3_buf.at[slot],
            c_sems.at[slot],
        ).start()

    def contig_wait(slot):
        pltpu.make_async_copy(
            o_in.at[0, pl.ds(0, _BM), :, :], a3_buf.at[slot],
            c_sems.at[slot],
        ).wait()

    def transpose(slot, b):
        cps = [
            pltpu.make_async_copy(
                a3_buf.at[slot, :, h, :],
                a_buf.at[b, :, pl.ds(h * _HD, _HD)],
                a_sems.at[slot],
            )
            for h in range(_NH)
        ]
        for cp in cps:
            cp.start()
        for cp in cps:
            cp.wait()

    def slc(b, ch, q):
        return o_ref.at[b, q, :, pl.ds(ch * _BN, _BN)]

    def rd(src, dst, ssem, rsem, k, dev):
        return pltpu.make_async_remote_copy(
            src_ref=src, dst_ref=dst, send_sem=ssem.at[k], recv_sem=rsem.at[k],
            device_id=dev, device_id_type=pl.DeviceIdType.MESH,
        )

    def rd_y(k, slot, b, ch):
        return rd(send_buf.at[slot], slc(b, ch, my_z), y_s, y_r, k, nbr_y)

    def rd_x(k, b, ch):
        return rd(slc(b, ch, my_z), slc(b, ch, my_z), x_s, x_r, k, nbr_x)

    def rd_z(k, b, ch):
        return rd(slc(b, ch, my_z), slc(b, ch, my_z), z_s, z_r, k, nbr_z)

    def rd_z_recv(k, b, ch):
        return rd(slc(b, ch, my_z), slc(b, ch, 1 - my_z), z_s, z_r, k, nbr_z)

    def rd_f(k, b, ch):
        return rd(slc(b, ch, my_z), slc(b, ch, my_z), f_s, f_r, k, nbr_z)

    def rd_f_recv(k, b, ch):
        return rd(slc(b, ch, my_z), slc(b, ch, 1 - my_z), f_s, f_r, k, nbr_z)

    def wo_start(ch, par):
        pltpu.make_async_copy(
            wo_ref.at[:, pl.ds(ch * _BN, _BN)], wo_bufs.at[par],
            wo_sems.at[par],
        ).start()

    def wo_wait(ch, par):
        pltpu.make_async_copy(
            wo_ref.at[:, pl.ds(ch * _BN, _BN)], wo_bufs.at[par],
            wo_sems.at[par],
        ).wait()

    wo_start(0, 0)
    contig_start(0, 0)
    barrier = pltpu.get_barrier_semaphore()
    for nbr in (nbr_y, nbr_x, nbr_z):
        pl.semaphore_signal(barrier, inc=1, device_id=nbr,
                            device_id_type=pl.DeviceIdType.MESH)
    pl.semaphore_wait(barrier, 3)

    for ch in range(_NCH):
        par = ch % 2
        wo_wait(ch, par)
        if ch + 1 < _NCH:
            wo_start(ch + 1, 1 - par)

        for b in range(_NB):
            k = ch * _NB + b
            slot = k % 2
            if ch == 0:
                if b + 1 < _NB:
                    contig_start(b + 1, (b + 1) % 2)
                contig_wait(b % 2)
                transpose(b % 2, b)
            v = jnp.dot(a_buf[b], wo_bufs[par],
                        preferred_element_type=jnp.float32)

            @pl.when(jnp.logical_not(is_holder))
            def _(v=v, k=k, slot=slot, b=b, ch=ch):
                if k >= 2:
                    rd_y(k - 2, slot, b, ch).wait_send()
                send_buf[slot] = v
                rd_y(k, slot, b, ch).start()
                if k >= 2:
                    j = k - 2
                    jb, jch = j % _NB, j // _NB
                    rd_x(j, jb, jch).wait_recv()
                    rd_f(j, jb, jch).start()

            @pl.when(is_holder)
            def _(v=v, k=k, slot=slot, b=b, ch=ch):
                rd_y(k, slot, b, ch).wait_recv()
                o_ref[b, my_z, :, pl.ds(ch * _BN, _BN)] = (
                    o_ref[b, my_z, :, pl.ds(ch * _BN, _BN)] + v
                )
                rd_x(k, b, ch).start()
                rd_z(k, b, ch).start()

    for k in (_NK - 2, _NK - 1):
        b, ch = k % _NB, k // _NB

        @pl.when(jnp.logical_not(is_holder))
        def _(k=k, b=b, ch=ch):
            rd_x(k, b, ch).wait_recv()
            rd_f(k, b, ch).start()

    for k in range(_NK):
        b, ch = k % _NB, k // _NB

        @pl.when(is_holder)
        def _(k=k, b=b, ch=ch):
            rd_z_recv(k, b, ch).wait_recv()

        @pl.when(jnp.logical_not(is_holder))
        def _(k=k, b=b, ch=ch):
            rd_f_recv(k, b, ch).wait_recv()

    @pl.when(is_holder)
    def _():
        for k in range(_NK):
            b, ch = k % _NB, k // _NB
            rd_x(k, b, ch).wait_send()
            rd_z(k, b, ch).wait_send()

    @pl.when(jnp.logical_not(is_holder))
    def _():
        for k in (_NK - 2, _NK - 1):
            rd_y(k, k % 2, k % _NB, k // _NB).wait_send()
        for k in range(_NK):
            b, ch = k % _NB, k // _NB
            rd_f(k, b, ch).wait_send()

    def _exit(second_barrier):
        for nbr in (nbr_y, nbr_x, nbr_z):
            pl.semaphore_signal(second_barrier, inc=1, device_id=nbr,
                                device_id_type=pl.DeviceIdType.MESH)
        pl.semaphore_wait(second_barrier, 3)

    pl.run_scoped(_exit, second_barrier=pltpu.SemaphoreType.REGULAR)


def kernel(O, Wo):
    B, S, Hl, D = O.shape
    N = Wo.shape[1]
    out = pl.pallas_call(
        _body,
        out_shape=jax.ShapeDtypeStruct((B, 2, _BM, N), jnp.float32),
        in_specs=[
            pl.BlockSpec(memory_space=pl.ANY),
            pl.BlockSpec(memory_space=pl.ANY),
        ],
        out_specs=pl.BlockSpec(memory_space=pltpu.VMEM),
        scratch_shapes=[
            pltpu.VMEM((2, Hl * D, _BN), jnp.float32),
            pltpu.VMEM((2, _BM, Hl, D), jnp.float32),
            pltpu.VMEM((B, _BM, Hl * D), jnp.float32),
            pltpu.VMEM((2, _BM, _BN), jnp.float32),
            pltpu.SemaphoreType.DMA((2,)),
            pltpu.SemaphoreType.DMA((2,)),
            pltpu.SemaphoreType.DMA((2,)),
        ] + [pltpu.SemaphoreType.DMA((_NK,)) for _ in range(8)],
        compiler_params=pltpu.CompilerParams(
            collective_id=0,
            vmem_limit_bytes=62 * 1024 * 1024,
        ),
    )(O, Wo)
    return out.reshape(B, 2 * _BM, N)
